# baseline (speedup 1.0000x reference)
import functools

import jax
import jax.numpy as jnp
from jax import lax
from jax.experimental import pallas as pl
from jax.experimental.pallas import tpu as pltpu
from jax.experimental.pallas import tpu_sc as plsc

DIM = 64
NBUF = 4


@functools.cache
def _make_gather(R, S, nw, nc):
    rpw = R // nw
    mesh = plsc.VectorSubcoreMesh(core_axis_name="c", subcore_axis_name="s")
    assert rpw % NBUF == 0

    @functools.partial(
        pl.kernel,
        mesh=mesh,
        compiler_params=pltpu.CompilerParams(use_tc_tiling_on_sc=False),
        out_type=jax.ShapeDtypeStruct((R, S, DIM), jnp.float32),
        scratch_types=[
            pltpu.VMEM((rpw, S), jnp.int32),
            pltpu.VMEM((NBUF, S, DIM), jnp.float32),
            pltpu.SemaphoreType.DMA((NBUF,)),
            pltpu.SemaphoreType.DMA((NBUF,)),
        ],
    )
    def gather_kernel(table_hbm, idx_hbm, out_hbm, idx_v, rows_v, gsem, ssem):
        wid = lax.axis_index("s") * nc + lax.axis_index("c")
        base = wid * rpw
        pltpu.sync_copy(idx_hbm.at[wid], idx_v)

        for b in range(NBUF):
            pltpu.async_copy(table_hbm.at[idx_v.at[b]], rows_v.at[b], gsem.at[b])

        def body(jj, carry):
            j0 = jj * NBUF
            for b in range(NBUF):
                j = j0 + b
                pltpu.make_async_copy(
                    table_hbm.at[idx_v.at[j]], rows_v.at[b], gsem.at[b]
                ).wait()
                pltpu.async_copy(rows_v.at[b], out_hbm.at[base + j], ssem.at[b])
            for b in range(NBUF):
                jn = j0 + NBUF + b
                pltpu.make_async_copy(
                    rows_v.at[b], out_hbm.at[base], ssem.at[b]
                ).wait()

                @pl.when(jn < rpw)
                def _():
                    pltpu.async_copy(table_hbm.at[idx_v.at[jn]], rows_v.at[b], gsem.at[b])

            return carry

        lax.fori_loop(0, rpw // NBUF, body, 0)

    return gather_kernel


def kernel(x, table_en, table_zh, table_jp):
    table = jnp.concatenate([table_en, table_zh, table_jp], axis=0)
    info = plsc.get_sparse_core_info()
    nw = info.num_cores * info.num_subcores
    R, S = x.shape
    idx3 = x.reshape(nw, R // nw, S)
    return _make_gather(R, S, nw, info.num_cores)(table, idx3)
